# baseline (speedup 1.0000x reference)
import functools
import jax
import jax.numpy as jnp
from jax import lax
from jax.experimental import pallas as pl
from jax.experimental.pallas import tpu as pltpu
from jax.experimental.pallas import tpu_sc as plsc

_NUM_RANKS = 512
_MAX_TOKENS = 77
_DIM = 768
_C = 16
_K = 4
_NW = 32
_RB = _NUM_RANKS // _NW
_CTX_REP = 16
_NRING = 3


def _sc_body(cbc_hbm, rankt_hbm, sentt_hbm, out_hbm, bufs, bbuf,
             rsems, wsems, csem, cwsem):
    nc = 2
    wid = lax.axis_index("s") * nc + lax.axis_index("c")
    ctx_row = wid % _C
    half = wid // _C
    rb = pl.ds(wid * _RB, _RB)

    cread = pltpu.make_async_copy(cbc_hbm.at[ctx_row], bbuf, csem)
    cread.start()

    chunks = [(0, 1), (1 + _C, 2), (3 + _C, 2)]
    chunks += [(t, min(3, _MAX_TOKENS - t))
               for t in range(1 + _C + _K, _MAX_TOKENS, 3)]

    def src(t, m):
        if t == 0:
            return sentt_hbm.at[pl.ds(0, m), rb]
        if t < 1 + _C + _K:
            return rankt_hbm.at[pl.ds(t - 1 - _C, m), rb]
        return sentt_hbm.at[pl.ds(t, m), rb]

    def read(n):
        t, m = chunks[n]
        return pltpu.make_async_copy(
            src(t, m), bufs.at[n % _NRING, pl.ds(0, m)],
            rsems.at[n % _NRING])

    def write(n):
        t, m = chunks[n]
        return pltpu.make_async_copy(
            bufs.at[n % _NRING, pl.ds(0, m)],
            out_hbm.at[pl.ds(t, m), rb],
            wsems.at[n % _NRING])

    cread.wait()
    n_fan = _NUM_RANKS // 2 // _CTX_REP
    cwrites = [
        pltpu.make_async_copy(
            bbuf,
            out_hbm.at[1 + ctx_row,
                       pl.ds(half * (_NUM_RANKS // 2) + k * _CTX_REP,
                             _CTX_REP)],
            cwsem)
        for k in range(n_fan)
    ]
    for cw in cwrites:
        cw.start()

    n_rows = len(chunks)
    for n in range(n_rows):
        if n >= _NRING:
            write(n - _NRING).wait()
        read(n).start()
        if n >= 1:
            read(n - 1).wait()
            write(n - 1).start()
    read(n_rows - 1).wait()
    write(n_rows - 1).start()
    for n in range(n_rows - _NRING, n_rows):
        write(n).wait()
    for cw in cwrites:
        cw.wait()


def kernel(context_embeds, rank_embeds, sentence_embeds):
    dt = sentence_embeds.dtype
    sent_t = jnp.transpose(sentence_embeds, (1, 0, 2))
    rank_t = jnp.transpose(rank_embeds, (1, 0, 2))
    cbc = jnp.broadcast_to(
        context_embeds[:, None, :], (_C, _CTX_REP, _DIM))
    mesh = plsc.VectorSubcoreMesh(core_axis_name="c", subcore_axis_name="s")
    k = functools.partial(
        pl.kernel,
        mesh=mesh,
        out_type=jax.ShapeDtypeStruct((_MAX_TOKENS, _NUM_RANKS, _DIM), dt),
        scratch_types=[
            pltpu.VMEM((_NRING, 3, _RB, _DIM), dt),
            pltpu.VMEM((_CTX_REP, _DIM), dt),
            pltpu.SemaphoreType.DMA((_NRING,)),
            pltpu.SemaphoreType.DMA((_NRING,)),
            pltpu.SemaphoreType.DMA,
            pltpu.SemaphoreType.DMA,
        ],
    )(_sc_body)
    out_t = k(cbc, rank_t, sent_t)
    return jnp.transpose(out_t, (1, 0, 2))
